# baseline (speedup 1.0000x reference)
import functools

import jax
import jax.numpy as jnp
from jax import lax
from jax.experimental import pallas as pl
from jax.experimental.pallas import tpu as pltpu
from jax.experimental.pallas import tpu_sc as plsc

B = 16384
N = 1000000
D = 64
LAT = 128
NC = 2
NS = 16
NW = NC * NS
BPW = B // NW
ICH = 128
NCH = BPW // ICH
BC = 32768
GRID = (N + BC - 1) // BC


def _tv_body(uTt_ref, iTt_ref, uA_ref, iA_ref, w_ref, tvu_ref, tvi_ref):
    w = w_ref[...]
    wu = jnp.dot(uA_ref[...], w[:LAT, 0], preferred_element_type=jnp.float32)
    wi = jnp.dot(iA_ref[...], w[LAT:, 0], preferred_element_type=jnp.float32)
    tvu_ref[...] = jnp.dot(wu, uTt_ref[...],
                           preferred_element_type=jnp.float32)
    tvi_ref[...] = jnp.dot(wi, iTt_ref[...],
                           preferred_element_type=jnp.float32)


def _tv(uTt, iTt, user_A, item_A, W_aff):
    return pl.pallas_call(
        _tv_body,
        grid=(GRID,),
        in_specs=[
            pl.BlockSpec((D, BC), lambda i: (0, i)),
            pl.BlockSpec((D, BC), lambda i: (0, i)),
            pl.BlockSpec((D, LAT), lambda i: (0, 0)),
            pl.BlockSpec((D, LAT), lambda i: (0, 0)),
            pl.BlockSpec((2 * LAT, 1), lambda i: (0, 0)),
        ],
        out_specs=[
            pl.BlockSpec((BC,), lambda i: (i,)),
            pl.BlockSpec((BC,), lambda i: (i,)),
        ],
        out_shape=[
            jax.ShapeDtypeStruct((N,), jnp.float32),
            jax.ShapeDtypeStruct((N,), jnp.float32),
        ],
    )(uTt, iTt, user_A, item_A, W_aff)


def _gather_body(uidx_h, iidx_h, tvu_h, tvi_h, b_h, out_h,
                 uidx_v, iidx_v, bu_v, bi_v, b_v, out_v, gsem):
    cid = lax.axis_index("c")
    sid = lax.axis_index("s")
    wid = sid * NC + cid
    base = wid * BPW

    pltpu.sync_copy(uidx_h.at[wid], uidx_v)
    pltpu.sync_copy(iidx_h.at[wid], iidx_v)
    pltpu.sync_copy(b_h, b_v)
    copies = []
    for j in range(NCH):
        copies.append(pltpu.async_copy(
            tvu_h.at[uidx_v.at[j]], bu_v.at[pl.ds(j * ICH, ICH)], gsem))
        copies.append(pltpu.async_copy(
            tvi_h.at[iidx_v.at[j]], bi_v.at[pl.ds(j * ICH, ICH)], gsem))
    for cp in copies:
        cp.wait()

    bias = b_v[...]

    def step(t, _):
        off = pl.multiple_of(t * 16, 16)
        out_v[pl.ds(off, 16)] = bu_v[pl.ds(off, 16)] + bi_v[pl.ds(off, 16)] + bias
        return 0

    lax.fori_loop(0, BPW // 16, step, 0)
    pltpu.sync_copy(out_v, out_h.at[pl.ds(base, BPW)])


@functools.partial(
    pl.kernel,
    out_type=jax.ShapeDtypeStruct((B,), jnp.float32),
    mesh=plsc.VectorSubcoreMesh(core_axis_name="c", subcore_axis_name="s"),
    compiler_params=pltpu.CompilerParams(use_tc_tiling_on_sc=False),
    scratch_types=[
        pltpu.VMEM((NCH, ICH), jnp.int32),
        pltpu.VMEM((NCH, ICH), jnp.int32),
        pltpu.VMEM((BPW,), jnp.float32),
        pltpu.VMEM((BPW,), jnp.float32),
        pltpu.VMEM((16,), jnp.float32),
        pltpu.VMEM((BPW,), jnp.float32),
        pltpu.SemaphoreType.DMA,
    ],
)
def _sc_gather(uidx_h, iidx_h, tvu_h, tvi_h, b_h, out_h, *scratch):
    _gather_body(uidx_h, iidx_h, tvu_h, tvi_h, b_h, out_h, *scratch)


def kernel(user_indices, item_indices, user_T, item_T, user_A, item_A,
           W_aff, b_aff):
    tvu, tvi = _tv(user_T.T, item_T.T, user_A, item_A, W_aff)
    uidx = user_indices.astype(jnp.int32).reshape(NW, NCH, ICH)
    iidx = item_indices.astype(jnp.int32).reshape(NW, NCH, ICH)
    b16 = jnp.broadcast_to(b_aff, (16,))
    out = _sc_gather(uidx, iidx, tvu, tvi, b16)
    return out.reshape(B, 1)


# speedup vs baseline: 1.0174x; 1.0174x over previous
import functools

import jax
import jax.numpy as jnp
from jax import lax
from jax.experimental import pallas as pl
from jax.experimental.pallas import tpu as pltpu
from jax.experimental.pallas import tpu_sc as plsc

B = 16384
N = 1000000
D = 64
LAT = 128
NC = 2
NS = 16
NW = NC * NS
BPW = B // NW
ICH = 128
NCH = BPW // ICH
BC = 16384
GRID = (N + BC - 1) // BC


def _tv_body(uTt_ref, iTt_ref, uA_ref, iA_ref, w_ref, tvu_ref, tvi_ref):
    w = w_ref[...]
    wu = jnp.dot(uA_ref[...], w[:LAT, 0], preferred_element_type=jnp.float32)
    wi = jnp.dot(iA_ref[...], w[LAT:, 0], preferred_element_type=jnp.float32)
    tvu_ref[...] = jnp.dot(wu, uTt_ref[...],
                           preferred_element_type=jnp.float32)
    tvi_ref[...] = jnp.dot(wi, iTt_ref[...],
                           preferred_element_type=jnp.float32)


def _tv(uTt, iTt, user_A, item_A, W_aff):
    return pl.pallas_call(
        _tv_body,
        grid=(GRID,),
        in_specs=[
            pl.BlockSpec((D, BC), lambda i: (0, i)),
            pl.BlockSpec((D, BC), lambda i: (0, i)),
            pl.BlockSpec((D, LAT), lambda i: (0, 0)),
            pl.BlockSpec((D, LAT), lambda i: (0, 0)),
            pl.BlockSpec((2 * LAT, 1), lambda i: (0, 0)),
        ],
        out_specs=[
            pl.BlockSpec((BC,), lambda i: (i,)),
            pl.BlockSpec((BC,), lambda i: (i,)),
        ],
        out_shape=[
            jax.ShapeDtypeStruct((N,), jnp.float32),
            jax.ShapeDtypeStruct((N,), jnp.float32),
        ],
    )(uTt, iTt, user_A, item_A, W_aff)


def _gather_body(uidx_h, iidx_h, tvu_h, tvi_h, b_h, out_h,
                 uidx_v, iidx_v, bu_v, bi_v, b_v, out_v, gsem):
    cid = lax.axis_index("c")
    sid = lax.axis_index("s")
    wid = sid * NC + cid
    base = wid * BPW

    pltpu.sync_copy(uidx_h.at[wid], uidx_v)
    pltpu.sync_copy(iidx_h.at[wid], iidx_v)
    pltpu.sync_copy(b_h, b_v)
    copies = []
    for j in range(NCH):
        copies.append(pltpu.async_copy(
            tvu_h.at[uidx_v.at[j]], bu_v.at[pl.ds(j * ICH, ICH)], gsem))
        copies.append(pltpu.async_copy(
            tvi_h.at[iidx_v.at[j]], bi_v.at[pl.ds(j * ICH, ICH)], gsem))
    for cp in copies:
        cp.wait()

    bias = b_v[...]

    def step(t, _):
        off = pl.multiple_of(t * 16, 16)
        out_v[pl.ds(off, 16)] = bu_v[pl.ds(off, 16)] + bi_v[pl.ds(off, 16)] + bias
        return 0

    lax.fori_loop(0, BPW // 16, step, 0)
    pltpu.sync_copy(out_v, out_h.at[pl.ds(base, BPW)])


@functools.partial(
    pl.kernel,
    out_type=jax.ShapeDtypeStruct((B,), jnp.float32),
    mesh=plsc.VectorSubcoreMesh(core_axis_name="c", subcore_axis_name="s"),
    compiler_params=pltpu.CompilerParams(use_tc_tiling_on_sc=False),
    scratch_types=[
        pltpu.VMEM((NCH, ICH), jnp.int32),
        pltpu.VMEM((NCH, ICH), jnp.int32),
        pltpu.VMEM((BPW,), jnp.float32),
        pltpu.VMEM((BPW,), jnp.float32),
        pltpu.VMEM((16,), jnp.float32),
        pltpu.VMEM((BPW,), jnp.float32),
        pltpu.SemaphoreType.DMA,
    ],
)
def _sc_gather(uidx_h, iidx_h, tvu_h, tvi_h, b_h, out_h, *scratch):
    _gather_body(uidx_h, iidx_h, tvu_h, tvi_h, b_h, out_h, *scratch)


def kernel(user_indices, item_indices, user_T, item_T, user_A, item_A,
           W_aff, b_aff):
    tvu, tvi = _tv(user_T.T, item_T.T, user_A, item_A, W_aff)
    uidx = user_indices.astype(jnp.int32).reshape(NW, NCH, ICH)
    iidx = item_indices.astype(jnp.int32).reshape(NW, NCH, ICH)
    b16 = jnp.broadcast_to(b_aff, (16,))
    out = _sc_gather(uidx, iidx, tvu, tvi, b16)
    return out.reshape(B, 1)


# speedup vs baseline: 1.0324x; 1.0148x over previous
import functools

import jax
import jax.numpy as jnp
from jax import lax
from jax.experimental import pallas as pl
from jax.experimental.pallas import tpu as pltpu
from jax.experimental.pallas import tpu_sc as plsc

B = 16384
N = 1000000
D = 64
LAT = 128
NC = 2
NS = 16
NW = NC * NS
BPW = B // NW
ICH = 128
NCH = BPW // ICH
BC = 20480
GRID = (N + BC - 1) // BC


def _tv_body(uTt_ref, iTt_ref, uA_ref, iA_ref, w_ref, tvu_ref, tvi_ref):
    w = w_ref[...]
    wu = jnp.dot(uA_ref[...], w[:LAT, 0], preferred_element_type=jnp.float32)
    wi = jnp.dot(iA_ref[...], w[LAT:, 0], preferred_element_type=jnp.float32)
    tvu_ref[...] = jnp.dot(wu, uTt_ref[...],
                           preferred_element_type=jnp.float32)
    tvi_ref[...] = jnp.dot(wi, iTt_ref[...],
                           preferred_element_type=jnp.float32)


def _tv(uTt, iTt, user_A, item_A, W_aff):
    return pl.pallas_call(
        _tv_body,
        grid=(GRID,),
        in_specs=[
            pl.BlockSpec((D, BC), lambda i: (0, i)),
            pl.BlockSpec((D, BC), lambda i: (0, i)),
            pl.BlockSpec((D, LAT), lambda i: (0, 0)),
            pl.BlockSpec((D, LAT), lambda i: (0, 0)),
            pl.BlockSpec((2 * LAT, 1), lambda i: (0, 0)),
        ],
        out_specs=[
            pl.BlockSpec((BC,), lambda i: (i,)),
            pl.BlockSpec((BC,), lambda i: (i,)),
        ],
        out_shape=[
            jax.ShapeDtypeStruct((N,), jnp.float32),
            jax.ShapeDtypeStruct((N,), jnp.float32),
        ],
    )(uTt, iTt, user_A, item_A, W_aff)


def _gather_body(uidx_h, iidx_h, tvu_h, tvi_h, b_h, out_h,
                 uidx_v, iidx_v, bu_v, bi_v, b_v, out_v, gsem):
    cid = lax.axis_index("c")
    sid = lax.axis_index("s")
    wid = sid * NC + cid
    base = wid * BPW

    pltpu.sync_copy(uidx_h.at[wid], uidx_v)
    pltpu.sync_copy(iidx_h.at[wid], iidx_v)
    pltpu.sync_copy(b_h, b_v)
    copies = []
    for j in range(NCH):
        copies.append(pltpu.async_copy(
            tvu_h.at[uidx_v.at[j]], bu_v.at[pl.ds(j * ICH, ICH)], gsem))
        copies.append(pltpu.async_copy(
            tvi_h.at[iidx_v.at[j]], bi_v.at[pl.ds(j * ICH, ICH)], gsem))
    for cp in copies:
        cp.wait()

    bias = b_v[...]

    def step(t, _):
        off = pl.multiple_of(t * 16, 16)
        out_v[pl.ds(off, 16)] = bu_v[pl.ds(off, 16)] + bi_v[pl.ds(off, 16)] + bias
        return 0

    lax.fori_loop(0, BPW // 16, step, 0)
    pltpu.sync_copy(out_v, out_h.at[pl.ds(base, BPW)])


@functools.partial(
    pl.kernel,
    out_type=jax.ShapeDtypeStruct((B,), jnp.float32),
    mesh=plsc.VectorSubcoreMesh(core_axis_name="c", subcore_axis_name="s"),
    compiler_params=pltpu.CompilerParams(use_tc_tiling_on_sc=False),
    scratch_types=[
        pltpu.VMEM((NCH, ICH), jnp.int32),
        pltpu.VMEM((NCH, ICH), jnp.int32),
        pltpu.VMEM((BPW,), jnp.float32),
        pltpu.VMEM((BPW,), jnp.float32),
        pltpu.VMEM((16,), jnp.float32),
        pltpu.VMEM((BPW,), jnp.float32),
        pltpu.SemaphoreType.DMA,
    ],
)
def _sc_gather(uidx_h, iidx_h, tvu_h, tvi_h, b_h, out_h, *scratch):
    _gather_body(uidx_h, iidx_h, tvu_h, tvi_h, b_h, out_h, *scratch)


def kernel(user_indices, item_indices, user_T, item_T, user_A, item_A,
           W_aff, b_aff):
    tvu, tvi = _tv(user_T.T, item_T.T, user_A, item_A, W_aff)
    uidx = user_indices.astype(jnp.int32).reshape(NW, NCH, ICH)
    iidx = item_indices.astype(jnp.int32).reshape(NW, NCH, ICH)
    b16 = jnp.broadcast_to(b_aff, (16,))
    out = _sc_gather(uidx, iidx, tvu, tvi, b16)
    return out.reshape(B, 1)
